# Initial kernel scaffold; baseline (speedup 1.0000x reference)
import jax
import jax.numpy as jnp
from jax.experimental import pallas as pl


def kernel(x, source_pos, target_pos):
    raise NotImplementedError("write your pallas kernel here")


# speedup vs baseline: 1.6606x; 1.6606x over previous
import functools

import jax
import jax.numpy as jnp
from jax import lax
from jax.experimental import pallas as pl
from jax.experimental.pallas import tpu as pltpu
from jax.experimental.pallas import tpu_sc as plsc

K = 8
EPS = 1e-08

B, C, N_IN, N_OUT = 4, 256, 2562, 10242
BC = B * C
N_PAD = 10752
N_INP = 2688
BT = 256
NW = 32
ROWS_W = BC // NW
TCH = 512
NCH = N_PAD // TCH
NG = TCH // 16


def _knn_kernel(tp_ref, sp_ref, idx_ref, w_ref):
    tp = tp_ref[...]
    sp = sp_ref[...]
    tx, ty, tz = tp[:, 0:1], tp[:, 1:2], tp[:, 2:3]
    sx, sy, sz = sp[0:1, :], sp[1:2, :], sp[2:3, :]
    q2 = (tx * tx + tz * tz) + ty * ty
    s2 = (sx * sx + sz * sz) + sy * sy
    mm = jnp.dot(tp, sp, preferred_element_type=jnp.float32)
    d2 = (q2 - 2.0 * mm) + s2
    iota = lax.broadcasted_iota(jnp.int32, (BT, N_INP), 1)
    idxs, ws = [], []
    wsum = jnp.zeros((BT, 1), jnp.float32)
    for _ in range(K):
        minv = jnp.min(d2, axis=1, keepdims=True)
        cand = jnp.where(d2 <= minv, iota, N_INP)
        idx = jnp.minimum(jnp.min(cand, axis=1, keepdims=True), N_IN - 1)
        d2 = jnp.where(iota == idx, jnp.float32(jnp.inf), d2)
        wk = 1.0 / (jnp.sqrt(jnp.maximum(minv, 0.0)) + EPS)
        idxs.append(idx)
        ws.append(wk)
        wsum = wsum + wk
    idx_ref[...] = jnp.concatenate(idxs, axis=1)
    w_ref[...] = jnp.concatenate(ws, axis=1) / wsum


def _stage1(tpos_pad, spos_pad):
    return pl.pallas_call(
        _knn_kernel,
        grid=(N_PAD // BT,),
        in_specs=[
            pl.BlockSpec((BT, 8), lambda i: (i, 0)),
            pl.BlockSpec((8, N_INP), lambda i: (0, 0)),
        ],
        out_specs=[
            pl.BlockSpec((BT, K), lambda i: (i, 0)),
            pl.BlockSpec((BT, K), lambda i: (i, 0)),
        ],
        out_shape=[
            jax.ShapeDtypeStruct((N_PAD, K), jnp.int32),
            jax.ShapeDtypeStruct((N_PAD, K), jnp.float32),
        ],
    )(tpos_pad, spos_pad)


@functools.cache
def _make_sc_gather():
    return functools.partial(
        pl.kernel,
        mesh=plsc.VectorSubcoreMesh(core_axis_name="c", subcore_axis_name="s"),
        out_type=jax.ShapeDtypeStruct((BC * N_PAD,), jnp.float32),
        scratch_types=[
            pltpu.VMEM((ROWS_W * N_INP,), jnp.float32),
            pltpu.VMEM((TCH * K,), jnp.int32),
            pltpu.VMEM((TCH * K,), jnp.float32),
            pltpu.VMEM((ROWS_W * TCH,), jnp.float32),
            pltpu.SemaphoreType.DMA,
        ],
        compiler_params=pltpu.CompilerParams(needs_layout_passes=False),
    )(_sc_gather_body)


def _sc_gather_body(xr_hbm, idx_hbm, w_hbm, out_hbm, xrows, idxc, wc, outc,
                    sem):
    wid = lax.axis_index("s") * 2 + lax.axis_index("c")
    r0 = wid * ROWS_W
    pltpu.sync_copy(xr_hbm.at[pl.ds(r0 * N_INP, ROWS_W * N_INP)], xrows)
    iota16 = lax.iota(jnp.int32, 16)

    def chunk_body(cix, _):
        c0 = cix * TCH
        pltpu.sync_copy(idx_hbm.at[pl.ds(c0 * K, TCH * K)], idxc)
        pltpu.sync_copy(w_hbm.at[pl.ds(c0 * K, TCH * K)], wc)

        def g_body(g, _):
            lanes = (g * 16 + iota16) * K
            idxs = [plsc.load_gather(idxc, [lanes + kk]) for kk in range(K)]
            wvs = [plsc.load_gather(wc, [lanes + kk]) for kk in range(K)]

            def r_body(r, _):
                rbase = r * N_INP
                acc = plsc.load_gather(xrows, [rbase + idxs[0]]) * wvs[0]
                for kk in range(1, K):
                    acc = acc + plsc.load_gather(xrows, [rbase + idxs[kk]]) * wvs[kk]
                outc[pl.ds(r * TCH + g * 16, 16)] = acc
                return 0

            lax.fori_loop(0, ROWS_W, r_body, 0)
            return 0

        lax.fori_loop(0, NG, g_body, 0)
        copies = [
            pltpu.async_copy(
                outc.at[pl.ds(r * TCH, TCH)],
                out_hbm.at[pl.ds((r0 + r) * N_PAD + c0, TCH)],
                sem,
            )
            for r in range(ROWS_W)
        ]
        for cp in copies:
            cp.wait()
        return 0

    lax.fori_loop(0, NCH, chunk_body, 0)


def kernel(x, source_pos, target_pos):
    xpad = jnp.pad(x.reshape(BC, N_IN), ((0, 0), (0, N_INP - N_IN)))
    tpos_pad = jnp.zeros((N_PAD, 8), jnp.float32).at[:N_OUT, :3].set(target_pos)
    spos_pad = (jnp.zeros((8, N_INP), jnp.float32)
                .at[:3, :].set(100.0)
                .at[:3, :N_IN].set(source_pos.T))
    idxp, wp = _stage1(tpos_pad, spos_pad)
    outp = _make_sc_gather()(
        xpad.reshape(BC * N_INP), idxp.reshape(N_PAD * K), wp.reshape(N_PAD * K)
    )
    return outp.reshape(BC, N_PAD)[:, :N_OUT].reshape(B, C, N_OUT)


# speedup vs baseline: 1.9072x; 1.1485x over previous
import functools

import jax
import jax.numpy as jnp
from jax import lax
from jax.experimental import pallas as pl
from jax.experimental.pallas import tpu as pltpu
from jax.experimental.pallas import tpu_sc as plsc

K = 8
EPS = 1e-08

B, C, N_IN, N_OUT = 4, 256, 2562, 10242
BC = B * C
N_PAD = 10752
N_INP = 2688
BT = 256
NW = 32
ROWS_W = BC // NW
TCH = 384
NCH = N_PAD // TCH
NG = TCH // 16


def _knn_kernel(tp_ref, sp_ref, idx_ref, w_ref):
    tp = tp_ref[...]
    sp = sp_ref[...]
    tx, ty, tz = tp[:, 0:1], tp[:, 1:2], tp[:, 2:3]
    sx, sy, sz = sp[0:1, :], sp[1:2, :], sp[2:3, :]
    q2 = (tx * tx + tz * tz) + ty * ty
    s2 = (sx * sx + sz * sz) + sy * sy
    mm = jnp.dot(tp, sp, preferred_element_type=jnp.float32)
    d2 = (q2 - 2.0 * mm) + s2
    iota = lax.broadcasted_iota(jnp.int32, (BT, N_INP), 1)
    idxs, ws = [], []
    wsum = jnp.zeros((BT, 1), jnp.float32)
    for _ in range(K):
        minv = jnp.min(d2, axis=1, keepdims=True)
        cand = jnp.where(d2 <= minv, iota, N_INP)
        idx = jnp.minimum(jnp.min(cand, axis=1, keepdims=True), N_IN - 1)
        d2 = jnp.where(iota == idx, jnp.float32(jnp.inf), d2)
        wk = 1.0 / (jnp.sqrt(jnp.maximum(minv, 0.0)) + EPS)
        idxs.append(idx)
        ws.append(wk)
        wsum = wsum + wk
    idx_ref[...] = jnp.concatenate(idxs, axis=1)
    w_ref[...] = jnp.concatenate(ws, axis=1) / wsum


def _stage1(tpos_pad, spos_pad):
    return pl.pallas_call(
        _knn_kernel,
        grid=(N_PAD // BT,),
        in_specs=[
            pl.BlockSpec((BT, 8), lambda i: (i, 0)),
            pl.BlockSpec((8, N_INP), lambda i: (0, 0)),
        ],
        out_specs=[
            pl.BlockSpec((BT, K), lambda i: (i, 0)),
            pl.BlockSpec((BT, K), lambda i: (i, 0)),
        ],
        out_shape=[
            jax.ShapeDtypeStruct((N_PAD, K), jnp.int32),
            jax.ShapeDtypeStruct((N_PAD, K), jnp.float32),
        ],
    )(tpos_pad, spos_pad)


@functools.cache
def _make_sc_gather():
    return functools.partial(
        pl.kernel,
        mesh=plsc.VectorSubcoreMesh(core_axis_name="c", subcore_axis_name="s"),
        out_type=jax.ShapeDtypeStruct((BC, N_PAD), jnp.float32),
        scratch_types=[
            pltpu.VMEM((ROWS_W * N_INP,), jnp.float32),
            pltpu.VMEM((TCH * K,), jnp.int32),
            pltpu.VMEM((TCH * K,), jnp.int32),
            pltpu.VMEM((TCH * K,), jnp.float32),
            pltpu.VMEM((TCH * K,), jnp.float32),
            pltpu.VMEM((ROWS_W, TCH), jnp.float32),
            pltpu.VMEM((ROWS_W, TCH), jnp.float32),
            pltpu.SemaphoreType.DMA,
            pltpu.SemaphoreType.DMA,
            pltpu.SemaphoreType.DMA,
            pltpu.SemaphoreType.DMA,
        ],
        compiler_params=pltpu.CompilerParams(needs_layout_passes=False),
    )(_sc_gather_body)


def _sc_gather_body(xr_hbm, idx_hbm, w_hbm, out_hbm,
                    xrows, idx_a, idx_b, w_a, w_b, out_a, out_b,
                    sin_a, sin_b, sout_a, sout_b):
    wid = lax.axis_index("s") * 2 + lax.axis_index("c")
    r0 = wid * ROWS_W
    pltpu.sync_copy(xr_hbm.at[pl.ds(r0 * N_INP, ROWS_W * N_INP)], xrows)
    iota16 = lax.iota(jnp.int32, 16)
    bufs = [(idx_a, w_a, out_a, sin_a, sout_a),
            (idx_b, w_b, out_b, sin_b, sout_b)]

    def fire_in(cix, p):
        idxc, wc, _, sin, _ = bufs[p]
        c0 = cix * TCH
        return (pltpu.async_copy(idx_hbm.at[pl.ds(c0 * K, TCH * K)], idxc, sin),
                pltpu.async_copy(w_hbm.at[pl.ds(c0 * K, TCH * K)], wc, sin))

    def wait_in(p):
        idxc, wc, _, sin, _ = bufs[p]
        pltpu.make_async_copy(idx_hbm.at[pl.ds(0, TCH * K)], idxc, sin).wait()
        pltpu.make_async_copy(w_hbm.at[pl.ds(0, TCH * K)], wc, sin).wait()

    def wait_out(p):
        _, _, outc, _, sout = bufs[p]
        pltpu.make_async_copy(
            outc, out_hbm.at[pl.ds(0, ROWS_W), pl.ds(0, TCH)], sout).wait()

    def compute(p, cix):
        idxc, wc, outc, _, sout = bufs[p]

        def g_body(g, _):
            lanes = (g * 16 + iota16) * K
            idxs = [plsc.load_gather(idxc, [lanes + kk]) for kk in range(K)]
            wvs = [plsc.load_gather(wc, [lanes + kk]) for kk in range(K)]
            col = g * 16

            def r_body(rq, _):
                for u in range(8):
                    r = rq * 8 + u
                    rbase = r * N_INP
                    acc = plsc.load_gather(xrows, [rbase + idxs[0]]) * wvs[0]
                    for kk in range(1, K):
                        acc = acc + plsc.load_gather(xrows, [rbase + idxs[kk]]) * wvs[kk]
                    outc[r, pl.ds(col, 16)] = acc
                return 0

            lax.fori_loop(0, ROWS_W // 8, r_body, 0)
            return 0

        lax.fori_loop(0, NG, g_body, 0)
        pltpu.async_copy(
            outc, out_hbm.at[pl.ds(r0, ROWS_W), pl.ds(cix * TCH, TCH)], sout)

    fire_in(0, 0)

    def pair_body(j, _):
        c_a = 2 * j
        wait_in(0)
        fire_in(c_a + 1, 1)
        pl.when(j >= 1)(lambda: None if wait_out(0) else None)
        compute(0, c_a)
        wait_in(1)
        pl.when(j + 1 < NCH // 2)(lambda: None if fire_in(c_a + 2, 0) else None)
        pl.when(j >= 1)(lambda: None if wait_out(1) else None)
        compute(1, c_a + 1)
        return 0

    lax.fori_loop(0, NCH // 2, pair_body, 0)
    wait_out(0)
    wait_out(1)


def kernel(x, source_pos, target_pos):
    xpad = jnp.pad(x.reshape(BC, N_IN), ((0, 0), (0, N_INP - N_IN)))
    tpos_pad = jnp.zeros((N_PAD, 8), jnp.float32).at[:N_OUT, :3].set(target_pos)
    spos_pad = (jnp.zeros((8, N_INP), jnp.float32)
                .at[:3, :].set(100.0)
                .at[:3, :N_IN].set(source_pos.T))
    idxp, wp = _stage1(tpos_pad, spos_pad)
    outp = _make_sc_gather()(
        xpad.reshape(BC * N_INP), idxp.reshape(N_PAD * K), wp.reshape(N_PAD * K)
    )
    return outp[:, :N_OUT].reshape(B, C, N_OUT)


# speedup vs baseline: 2.0540x; 1.0770x over previous
import functools

import jax
import jax.numpy as jnp
from jax import lax
from jax.experimental import pallas as pl
from jax.experimental.pallas import tpu as pltpu
from jax.experimental.pallas import tpu_sc as plsc

K = 8
EPS = 1e-08

B, C, N_IN, N_OUT = 4, 256, 2562, 10242
BC = B * C
N_PAD = 10752
N_INP = 2688
BT = 256
NW = 32
ROWS_W = BC // NW
TCH = 384
NCH = N_PAD // TCH
NG = TCH // 16


def _knn_kernel(tp_ref, sp_ref, idx_ref, w_ref):
    tp = tp_ref[...]
    sp = sp_ref[...]
    tx, ty, tz = tp[:, 0:1], tp[:, 1:2], tp[:, 2:3]
    sx, sy, sz = sp[0:1, :], sp[1:2, :], sp[2:3, :]
    q2 = (tx * tx + tz * tz) + ty * ty
    s2 = (sx * sx + sz * sz) + sy * sy
    mm = jnp.dot(tp, sp, preferred_element_type=jnp.float32)
    d2 = (q2 - 2.0 * mm) + s2
    iota = lax.broadcasted_iota(jnp.int32, (BT, N_INP), 1).astype(jnp.float32)
    idxs, ws = [], []
    wsum = jnp.zeros((BT, 1), jnp.float32)
    for _ in range(K):
        minv = jnp.min(d2, axis=1, keepdims=True)
        cand = jnp.where(d2 <= minv, iota, jnp.float32(N_INP))
        idxf = jnp.min(cand, axis=1, keepdims=True)
        d2 = jnp.where(iota == idxf, jnp.float32(jnp.inf), d2)
        idx = jnp.minimum(idxf.astype(jnp.int32), N_IN - 1)
        wk = 1.0 / (jnp.sqrt(jnp.maximum(minv, 0.0)) + EPS)
        idxs.append(idx)
        ws.append(wk)
        wsum = wsum + wk
    idx_ref[...] = jnp.concatenate(idxs, axis=1)
    w_ref[...] = jnp.concatenate(ws, axis=1) / wsum


def _stage1(tpos_pad, spos_pad):
    return pl.pallas_call(
        _knn_kernel,
        grid=(N_PAD // BT,),
        in_specs=[
            pl.BlockSpec((BT, 8), lambda i: (i, 0)),
            pl.BlockSpec((8, N_INP), lambda i: (0, 0)),
        ],
        out_specs=[
            pl.BlockSpec((BT, K), lambda i: (i, 0)),
            pl.BlockSpec((BT, K), lambda i: (i, 0)),
        ],
        out_shape=[
            jax.ShapeDtypeStruct((N_PAD, K), jnp.int32),
            jax.ShapeDtypeStruct((N_PAD, K), jnp.float32),
        ],
    )(tpos_pad, spos_pad)


@functools.cache
def _make_sc_gather():
    return functools.partial(
        pl.kernel,
        mesh=plsc.VectorSubcoreMesh(core_axis_name="c", subcore_axis_name="s"),
        out_type=jax.ShapeDtypeStruct((BC, N_PAD), jnp.float32),
        scratch_types=[
            pltpu.VMEM((ROWS_W * N_INP,), jnp.float32),
            pltpu.VMEM((TCH * K,), jnp.int32),
            pltpu.VMEM((TCH * K,), jnp.int32),
            pltpu.VMEM((TCH * K,), jnp.float32),
            pltpu.VMEM((TCH * K,), jnp.float32),
            pltpu.VMEM((ROWS_W, TCH), jnp.float32),
            pltpu.VMEM((ROWS_W, TCH), jnp.float32),
            pltpu.SemaphoreType.DMA,
            pltpu.SemaphoreType.DMA,
            pltpu.SemaphoreType.DMA,
            pltpu.SemaphoreType.DMA,
        ],
        compiler_params=pltpu.CompilerParams(needs_layout_passes=False),
    )(_sc_gather_body)


def _sc_gather_body(xr_hbm, idx_hbm, w_hbm, out_hbm,
                    xrows, idx_a, idx_b, w_a, w_b, out_a, out_b,
                    sin_a, sin_b, sout_a, sout_b):
    wid = lax.axis_index("s") * 2 + lax.axis_index("c")
    r0 = wid * ROWS_W
    pltpu.sync_copy(xr_hbm.at[pl.ds(r0 * N_INP, ROWS_W * N_INP)], xrows)
    iota16 = lax.iota(jnp.int32, 16)
    bufs = [(idx_a, w_a, out_a, sin_a, sout_a),
            (idx_b, w_b, out_b, sin_b, sout_b)]

    def fire_in(cix, p):
        idxc, wc, _, sin, _ = bufs[p]
        c0 = cix * TCH
        return (pltpu.async_copy(idx_hbm.at[pl.ds(c0 * K, TCH * K)], idxc, sin),
                pltpu.async_copy(w_hbm.at[pl.ds(c0 * K, TCH * K)], wc, sin))

    def wait_in(p):
        idxc, wc, _, sin, _ = bufs[p]
        pltpu.make_async_copy(idx_hbm.at[pl.ds(0, TCH * K)], idxc, sin).wait()
        pltpu.make_async_copy(w_hbm.at[pl.ds(0, TCH * K)], wc, sin).wait()

    def wait_out(p):
        _, _, outc, _, sout = bufs[p]
        pltpu.make_async_copy(
            outc, out_hbm.at[pl.ds(0, ROWS_W), pl.ds(0, TCH)], sout).wait()

    def compute(p, cix):
        idxc, wc, outc, _, sout = bufs[p]

        def g_body(g, _):
            lanes = (g * 16 + iota16) * K
            idxs = [plsc.load_gather(idxc, [lanes + kk]) for kk in range(K)]
            wvs = [plsc.load_gather(wc, [lanes + kk]) for kk in range(K)]
            col = g * 16

            def r_body(rq, _):
                for u in range(8):
                    r = rq * 8 + u
                    rbase = r * N_INP
                    acc = plsc.load_gather(xrows, [rbase + idxs[0]]) * wvs[0]
                    for kk in range(1, K):
                        acc = acc + plsc.load_gather(xrows, [rbase + idxs[kk]]) * wvs[kk]
                    outc[r, pl.ds(col, 16)] = acc
                return 0

            lax.fori_loop(0, ROWS_W // 8, r_body, 0)
            return 0

        lax.fori_loop(0, NG, g_body, 0)
        pltpu.async_copy(
            outc, out_hbm.at[pl.ds(r0, ROWS_W), pl.ds(cix * TCH, TCH)], sout)

    fire_in(0, 0)

    def pair_body(j, _):
        c_a = 2 * j
        wait_in(0)
        fire_in(c_a + 1, 1)
        pl.when(j >= 1)(lambda: None if wait_out(0) else None)
        compute(0, c_a)
        wait_in(1)
        pl.when(j + 1 < NCH // 2)(lambda: None if fire_in(c_a + 2, 0) else None)
        pl.when(j >= 1)(lambda: None if wait_out(1) else None)
        compute(1, c_a + 1)
        return 0

    lax.fori_loop(0, NCH // 2, pair_body, 0)
    wait_out(0)
    wait_out(1)


def kernel(x, source_pos, target_pos):
    xpad = jnp.pad(x.reshape(BC, N_IN), ((0, 0), (0, N_INP - N_IN)))
    tpos_pad = jnp.zeros((N_PAD, 8), jnp.float32).at[:N_OUT, :3].set(target_pos)
    spos_pad = (jnp.zeros((8, N_INP), jnp.float32)
                .at[:3, :].set(100.0)
                .at[:3, :N_IN].set(source_pos.T))
    idxp, wp = _stage1(tpos_pad, spos_pad)
    outp = _make_sc_gather()(
        xpad.reshape(BC * N_INP), idxp.reshape(N_PAD * K), wp.reshape(N_PAD * K)
    )
    return outp[:, :N_OUT].reshape(B, C, N_OUT)


# speedup vs baseline: 2.4084x; 1.1726x over previous
import functools

import jax
import jax.numpy as jnp
from jax import lax
from jax.experimental import pallas as pl
from jax.experimental.pallas import tpu as pltpu
from jax.experimental.pallas import tpu_sc as plsc

K = 8
EPS = 1e-08

B, C, N_IN, N_OUT = 4, 256, 2562, 10242
BC = B * C
N_PAD = 10752
N_INP = 2688
BT = 256
NW = 32
ROWS_W = BC // NW
TCH = 384
NCH = N_PAD // TCH
NG = TCH // 16


def _knn_kernel(tp_ref, sp_ref, idx_ref, w_ref):
    tp = tp_ref[...]
    sp = sp_ref[...]
    tx, ty, tz = tp[:, 0:1], tp[:, 1:2], tp[:, 2:3]
    sx, sy, sz = sp[0:1, :], sp[1:2, :], sp[2:3, :]
    q2 = (tx * tx + tz * tz) + ty * ty
    s2 = (sx * sx + sz * sz) + sy * sy
    mm = jnp.dot(tp, sp, preferred_element_type=jnp.float32)
    d2 = (q2 - 2.0 * mm) + s2
    iota = lax.broadcasted_iota(jnp.int32, (BT, N_INP), 1).astype(jnp.float32)
    idxs, ws = [], []
    wsum = jnp.zeros((BT, 1), jnp.float32)
    for _ in range(K):
        minv = jnp.min(d2, axis=1, keepdims=True)
        cand = jnp.where(d2 <= minv, iota, jnp.float32(N_INP))
        idxf = jnp.min(cand, axis=1, keepdims=True)
        d2 = jnp.where(iota == idxf, jnp.float32(jnp.inf), d2)
        idx = jnp.minimum(idxf.astype(jnp.int32), N_IN - 1)
        wk = 1.0 / (jnp.sqrt(jnp.maximum(minv, 0.0)) + EPS)
        idxs.append(idx)
        ws.append(wk)
        wsum = wsum + wk
    idx_ref[...] = jnp.concatenate(idxs, axis=1)
    w_ref[...] = jnp.concatenate(ws, axis=1) / wsum


def _stage1(tpos_pad, spos_pad):
    nt = tpos_pad.shape[0]
    return pl.pallas_call(
        _knn_kernel,
        grid=(nt // BT,),
        in_specs=[
            pl.BlockSpec((BT, 8), lambda i: (i, 0)),
            pl.BlockSpec((8, N_INP), lambda i: (0, 0)),
        ],
        out_specs=[
            pl.BlockSpec((BT, K), lambda i: (i, 0)),
            pl.BlockSpec((BT, K), lambda i: (i, 0)),
        ],
        out_shape=[
            jax.ShapeDtypeStruct((nt, K), jnp.int32),
            jax.ShapeDtypeStruct((nt, K), jnp.float32),
        ],
    )(tpos_pad, spos_pad)


@functools.cache
def _make_sc_gather(nt=N_PAD):
    nch = nt // TCH

    def _sc_gather_body(xr_hbm, idx_hbm, w_hbm, out_hbm,
                        xrows, idx_a, idx_b, w_a, w_b, out_a, out_b,
                        sin_a, sin_b, sout_a, sout_b):
        wid = lax.axis_index("s") * 2 + lax.axis_index("c")
        r0 = wid * ROWS_W
        pltpu.sync_copy(xr_hbm.at[pl.ds(r0 * N_INP, ROWS_W * N_INP)], xrows)
        iota16 = lax.iota(jnp.int32, 16)
        bufs = [(idx_a, w_a, out_a, sin_a, sout_a),
                (idx_b, w_b, out_b, sin_b, sout_b)]

        def fire_in(cix, p):
            idxc, wc, _, sin, _ = bufs[p]
            c0 = cix * TCH
            return (pltpu.async_copy(idx_hbm.at[pl.ds(c0 * K, TCH * K)], idxc, sin),
                    pltpu.async_copy(w_hbm.at[pl.ds(c0 * K, TCH * K)], wc, sin))

        def wait_in(p):
            idxc, wc, _, sin, _ = bufs[p]
            pltpu.make_async_copy(idx_hbm.at[pl.ds(0, TCH * K)], idxc, sin).wait()
            pltpu.make_async_copy(w_hbm.at[pl.ds(0, TCH * K)], wc, sin).wait()

        def wait_out(p):
            _, _, outc, _, sout = bufs[p]
            pltpu.make_async_copy(
                outc, out_hbm.at[pl.ds(0, ROWS_W), pl.ds(0, TCH)], sout).wait()

        def compute(p, cix):
            idxc, wc, outc, _, sout = bufs[p]

            def g_body(g, _):
                lanes = (g * 16 + iota16) * K
                idxs = [plsc.load_gather(idxc, [lanes + kk]) for kk in range(K)]
                wvs = [plsc.load_gather(wc, [lanes + kk]) for kk in range(K)]
                col = g * 16

                def r_body(rq, _):
                    for u in range(8):
                        r = rq * 8 + u
                        rbase = r * N_INP
                        acc = plsc.load_gather(xrows, [rbase + idxs[0]]) * wvs[0]
                        for kk in range(1, K):
                            acc = acc + plsc.load_gather(xrows, [rbase + idxs[kk]]) * wvs[kk]
                        outc[r, pl.ds(col, 16)] = acc
                    return 0

                lax.fori_loop(0, ROWS_W // 8, r_body, 0)
                return 0

            lax.fori_loop(0, NG, g_body, 0)
            pltpu.async_copy(
                outc, out_hbm.at[pl.ds(r0, ROWS_W), pl.ds(cix * TCH, TCH)], sout)

        fire_in(0, 0)

        def pair_body(j, _):
            c_a = 2 * j
            wait_in(0)
            fire_in(c_a + 1, 1)
            pl.when(j >= 1)(lambda: None if wait_out(0) else None)
            compute(0, c_a)
            wait_in(1)
            pl.when(j + 1 < nch // 2)(lambda: None if fire_in(c_a + 2, 0) else None)
            pl.when(j >= 1)(lambda: None if wait_out(1) else None)
            compute(1, c_a + 1)
            return 0

        lax.fori_loop(0, nch // 2, pair_body, 0)
        wait_out(0)
        wait_out(1)

    return functools.partial(
        pl.kernel,
        mesh=plsc.VectorSubcoreMesh(core_axis_name="c", subcore_axis_name="s"),
        out_type=jax.ShapeDtypeStruct((BC, nt), jnp.float32),
        scratch_types=[
            pltpu.VMEM((ROWS_W * N_INP,), jnp.float32),
            pltpu.VMEM((TCH * K,), jnp.int32),
            pltpu.VMEM((TCH * K,), jnp.int32),
            pltpu.VMEM((TCH * K,), jnp.float32),
            pltpu.VMEM((TCH * K,), jnp.float32),
            pltpu.VMEM((ROWS_W, TCH), jnp.float32),
            pltpu.VMEM((ROWS_W, TCH), jnp.float32),
            pltpu.SemaphoreType.DMA,
            pltpu.SemaphoreType.DMA,
            pltpu.SemaphoreType.DMA,
            pltpu.SemaphoreType.DMA,
        ],
        compiler_params=pltpu.CompilerParams(needs_layout_passes=False),
    )(_sc_gather_body)


N_H = N_PAD // 2


def kernel(x, source_pos, target_pos):
    xpad = jnp.pad(x.reshape(BC, N_IN), ((0, 0), (0, N_INP - N_IN)))
    xflat = xpad.reshape(BC * N_INP)
    tpos_pad = jnp.zeros((N_PAD, 8), jnp.float32).at[:N_OUT, :3].set(target_pos)
    spos_pad = (jnp.zeros((8, N_INP), jnp.float32)
                .at[:3, :].set(100.0)
                .at[:3, :N_IN].set(source_pos.T))
    sc = _make_sc_gather(N_H)
    idx0, w0 = _stage1(tpos_pad[:N_H], spos_pad)
    out0 = sc(xflat, idx0.reshape(N_H * K), w0.reshape(N_H * K))
    idx1, w1 = _stage1(tpos_pad[N_H:], spos_pad)
    out1 = sc(xflat, idx1.reshape(N_H * K), w1.reshape(N_H * K))
    outp = jnp.concatenate([out0, out1], axis=1)
    return outp[:, :N_OUT].reshape(B, C, N_OUT)
